# baseline (speedup 1.0000x reference)
import functools

import jax
import jax.numpy as jnp
from jax import lax
from jax.experimental import pallas as pl
from jax.experimental.pallas import tpu as pltpu
from jax.experimental.pallas import tpu_sc as plsc

_VOCAB = 1000000
_EMBED = 64
_B = 16384
_L = 20
_N = _B * _L
_NC = 2
_NS = 16
_NW = _NC * _NS
_ROWS_W = _B // _NW
_PER_W = _N // _NW
_XF_C = 64
_XF_R = _N // _XF_C
_XFR_W = _PER_W // _XF_C
_RBLK = 16
_CH = _RBLK * _L
_CH_XFR = _CH // _XF_C
_NCHUNK = _ROWS_W // _RBLK
_KT = 512

_mesh = plsc.VectorSubcoreMesh(core_axis_name="c", subcore_axis_name="s")


@functools.partial(
    pl.pallas_call,
    out_shape=jax.ShapeDtypeStruct((_VOCAB, 128), jnp.float32),
    grid=(pl.cdiv(_VOCAB, _KT),),
    in_specs=[pl.BlockSpec((_EMBED, _KT), lambda i: (0, i))],
    out_specs=pl.BlockSpec((_KT, 128), lambda i: (i, 0)),
)
def _pad_transpose_tc(wt_ref, out_ref):
    eye_pad = jnp.concatenate(
        [jnp.eye(_EMBED, dtype=jnp.float32),
         jnp.zeros((_EMBED, 128 - _EMBED), jnp.float32)], axis=1)
    out_ref[...] = jax.lax.dot_general(
        wt_ref[...], eye_pad, (((0,), (0,)), ((), ())),
        preferred_element_type=jnp.float32,
        precision=jax.lax.Precision.HIGHEST)


@functools.partial(
    pl.kernel,
    mesh=_mesh,
    out_type=jax.ShapeDtypeStruct((_XF_R, _XF_C), jnp.int32),
    scratch_types=[
        pltpu.VMEM((_ROWS_W, _L), jnp.int32),
        pltpu.VMEM((_XFR_W, _XF_C), jnp.int32),
    ],
    compiler_params=pltpu.CompilerParams(needs_layout_passes=False),
)
def _flatten_sc(x_hbm, xf_hbm, xv, fv):
    wid = lax.axis_index("s") * _NC + lax.axis_index("c")
    r0 = wid * _ROWS_W
    pltpu.sync_copy(x_hbm.at[pl.ds(r0, _ROWS_W), :], xv)

    def body(r, carry):
        i16 = lax.iota(jnp.int32, 16)
        p1 = r * _L + i16
        v1 = xv[r, pl.ds(0, 16)]
        plsc.store_scatter(fv, [p1 // _XF_C, p1 % _XF_C], v1)
        p2 = p1 + 4
        v2 = xv[r, pl.ds(4, 16)]
        plsc.store_scatter(fv, [p2 // _XF_C, p2 % _XF_C], v2, mask=i16 >= 12)
        return carry

    lax.fori_loop(0, _ROWS_W, body, 0)
    pltpu.sync_copy(fv, xf_hbm.at[pl.ds(wid * _XFR_W, _XFR_W), :])


_BLK_CUM = [(k + 1) * _XF_C // _L for k in range(_CH_XFR)]


@functools.partial(
    pl.kernel,
    mesh=_mesh,
    out_type=jax.ShapeDtypeStruct((_B, _L, _EMBED), jnp.float32),
    scratch_types=[
        pltpu.VMEM((_XFR_W, _XF_C), jnp.int32),
        pltpu.VMEM((_CH, 128), jnp.float32),
        pltpu.VMEM((_CH, _EMBED), jnp.float32),
        [pltpu.SemaphoreType.DMA] * _CH_XFR,
        pltpu.SemaphoreType.DMA,
    ],
)
def _embed_sc(xf_hbm, table_hbm, out_hbm, idx_v, rows_v, rows64_v, gsems, ssem):
    wid = lax.axis_index("s") * _NC + lax.axis_index("c")
    row0 = wid * _ROWS_W
    xfr0 = wid * _XFR_W
    pltpu.sync_copy(xf_hbm.at[pl.ds(xfr0, _XFR_W), :], idx_v)

    def body(i, carry):
        gds = [
            pltpu.async_copy(
                table_hbm.at[idx_v.at[i * _CH_XFR + k]],
                rows_v.at[pl.ds(k * _XF_C, _XF_C), :],
                gsems[k],
            )
            for k in range(_CH_XFR)
        ]
        r = row0 + i * _RBLK
        nblk = 0
        for k in range(_CH_XFR):
            gds[k].wait()

            def depad(j, c):
                for c4 in range(4):
                    rows64_v[j, pl.ds(c4 * 16, 16)] = rows_v[j, pl.ds(c4 * 16, 16)]
                return c

            lax.fori_loop(k * _XF_C, (k + 1) * _XF_C, depad, 0)
            for b in range(nblk, _BLK_CUM[k]):
                pltpu.async_copy(
                    rows64_v.at[pl.ds(b * _L, _L), :], out_hbm.at[r + b], ssem
                )
            nblk = _BLK_CUM[k]
        for b in range(_RBLK):
            pltpu.make_async_copy(
                rows64_v.at[pl.ds(b * _L, _L), :], out_hbm.at[r + b], ssem
            ).wait()
        return carry

    lax.fori_loop(0, _NCHUNK, body, 0)


def kernel(x, W):
    table = _pad_transpose_tc(W.T)
    xf = _flatten_sc(x.astype(jnp.int32))
    return _embed_sc(xf, table)


# speedup vs baseline: 2.2203x; 2.2203x over previous
import functools

import jax
import jax.numpy as jnp
from jax import lax
from jax.experimental import pallas as pl
from jax.experimental.pallas import tpu as pltpu
from jax.experimental.pallas import tpu_sc as plsc

_VOCAB = 1000000
_EMBED = 64
_B = 16384
_L = 20
_N = _B * _L
_NC = 2
_NS = 16
_NW = _NC * _NS
_ROWS_W = _B // _NW
_PER_W = _N // _NW
_XF_C = 64
_XF_R = _N // _XF_C
_XFR_W = _PER_W // _XF_C
_RBLK = 16
_CH = _RBLK * _L
_CH_XFR = _CH // _XF_C
_NCHUNK = _ROWS_W // _RBLK
_KT = 4096

_mesh = plsc.VectorSubcoreMesh(core_axis_name="c", subcore_axis_name="s")


@functools.partial(
    pl.pallas_call,
    out_shape=jax.ShapeDtypeStruct((_VOCAB, 128), jnp.float32),
    grid=(pl.cdiv(_VOCAB, _KT),),
    in_specs=[pl.BlockSpec((_EMBED, _KT), lambda i: (0, i))],
    out_specs=pl.BlockSpec((_KT, 128), lambda i: (i, 0)),
)
def _pad_transpose_tc(wt_ref, out_ref):
    eye_pad = jnp.concatenate(
        [jnp.eye(_EMBED, dtype=jnp.float32),
         jnp.zeros((_EMBED, 128 - _EMBED), jnp.float32)], axis=1)
    out_ref[...] = jax.lax.dot_general(
        wt_ref[...], eye_pad, (((0,), (0,)), ((), ())),
        preferred_element_type=jnp.float32,
        precision=jax.lax.Precision.HIGHEST)


@functools.partial(
    pl.kernel,
    mesh=_mesh,
    out_type=jax.ShapeDtypeStruct((_XF_R, _XF_C), jnp.int32),
    scratch_types=[
        pltpu.VMEM((_ROWS_W, _L), jnp.int32),
        pltpu.VMEM((_XFR_W, _XF_C), jnp.int32),
    ],
    compiler_params=pltpu.CompilerParams(needs_layout_passes=False),
)
def _flatten_sc(x_hbm, xf_hbm, xv, fv):
    wid = lax.axis_index("s") * _NC + lax.axis_index("c")
    r0 = wid * _ROWS_W
    pltpu.sync_copy(x_hbm.at[pl.ds(r0, _ROWS_W), :], xv)

    def body(r, carry):
        i16 = lax.iota(jnp.int32, 16)
        p1 = r * _L + i16
        v1 = xv[r, pl.ds(0, 16)]
        plsc.store_scatter(fv, [p1 // _XF_C, p1 % _XF_C], v1)
        p2 = p1 + 4
        v2 = xv[r, pl.ds(4, 16)]
        plsc.store_scatter(fv, [p2 // _XF_C, p2 % _XF_C], v2, mask=i16 >= 12)
        return carry

    lax.fori_loop(0, _ROWS_W, body, 0)
    pltpu.sync_copy(fv, xf_hbm.at[pl.ds(wid * _XFR_W, _XFR_W), :])


_BLK_CUM = [(k + 1) * _XF_C // _L for k in range(_CH_XFR)]


@functools.partial(
    pl.kernel,
    mesh=_mesh,
    out_type=jax.ShapeDtypeStruct((_B, _L, _EMBED), jnp.float32),
    scratch_types=[
        pltpu.VMEM((_XFR_W, _XF_C), jnp.int32),
        pltpu.VMEM((_CH, 128), jnp.float32),
        pltpu.VMEM((_CH, _EMBED), jnp.float32),
        [pltpu.SemaphoreType.DMA] * _CH_XFR,
        pltpu.SemaphoreType.DMA,
    ],
)
def _embed_sc(xf_hbm, table_hbm, out_hbm, idx_v, rows_v, rows64_v, gsems, ssem):
    wid = lax.axis_index("s") * _NC + lax.axis_index("c")
    row0 = wid * _ROWS_W
    xfr0 = wid * _XFR_W
    pltpu.sync_copy(xf_hbm.at[pl.ds(xfr0, _XFR_W), :], idx_v)

    def body(i, carry):
        gds = [
            pltpu.async_copy(
                table_hbm.at[idx_v.at[i * _CH_XFR + k]],
                rows_v.at[pl.ds(k * _XF_C, _XF_C), :],
                gsems[k],
            )
            for k in range(_CH_XFR)
        ]
        r = row0 + i * _RBLK
        nblk = 0
        for k in range(_CH_XFR):
            gds[k].wait()

            def depad(j, c):
                for c4 in range(4):
                    rows64_v[j, pl.ds(c4 * 16, 16)] = rows_v[j, pl.ds(c4 * 16, 16)]
                return c

            lax.fori_loop(k * _XF_C, (k + 1) * _XF_C, depad, 0)
            for b in range(nblk, _BLK_CUM[k]):
                pltpu.async_copy(
                    rows64_v.at[pl.ds(b * _L, _L), :], out_hbm.at[r + b], ssem
                )
            nblk = _BLK_CUM[k]
        for b in range(_RBLK):
            pltpu.make_async_copy(
                rows64_v.at[pl.ds(b * _L, _L), :], out_hbm.at[r + b], ssem
            ).wait()
        return carry

    lax.fori_loop(0, _NCHUNK, body, 0)


def kernel(x, W):
    table = _pad_transpose_tc(W.T)
    xf = _flatten_sc(x.astype(jnp.int32))
    return _embed_sc(xf, table)


# speedup vs baseline: 2.3643x; 1.0649x over previous
import functools

import jax
import jax.numpy as jnp
from jax import lax
from jax.experimental import pallas as pl
from jax.experimental.pallas import tpu as pltpu
from jax.experimental.pallas import tpu_sc as plsc

_VOCAB = 1000000
_EMBED = 64
_B = 16384
_L = 20
_N = _B * _L
_NC = 2
_NS = 16
_NW = _NC * _NS
_ROWS_W = _B // _NW
_PER_W = _N // _NW
_XF_C = 64
_XF_R = _N // _XF_C
_XFR_W = _PER_W // _XF_C
_RBLK = 16
_CH = _RBLK * _L
_CH_XFR = _CH // _XF_C
_NCHUNK = _ROWS_W // _RBLK
_KT = 8192

_mesh = plsc.VectorSubcoreMesh(core_axis_name="c", subcore_axis_name="s")


@functools.partial(
    pl.pallas_call,
    out_shape=jax.ShapeDtypeStruct((_VOCAB, 128), jnp.float32),
    grid=(pl.cdiv(_VOCAB, _KT),),
    in_specs=[pl.BlockSpec((_EMBED, _KT), lambda i: (0, i))],
    out_specs=pl.BlockSpec((_KT, 128), lambda i: (i, 0)),
)
def _pad_transpose_tc(wt_ref, out_ref):
    eye_pad = jnp.concatenate(
        [jnp.eye(_EMBED, dtype=jnp.float32),
         jnp.zeros((_EMBED, 128 - _EMBED), jnp.float32)], axis=1)
    out_ref[...] = jax.lax.dot_general(
        wt_ref[...], eye_pad, (((0,), (0,)), ((), ())),
        preferred_element_type=jnp.float32,
        precision=jax.lax.Precision.HIGHEST)


@functools.partial(
    pl.kernel,
    mesh=_mesh,
    out_type=jax.ShapeDtypeStruct((_XF_R, _XF_C), jnp.int32),
    scratch_types=[
        pltpu.VMEM((_ROWS_W, _L), jnp.int32),
        pltpu.VMEM((_XFR_W, _XF_C), jnp.int32),
    ],
    compiler_params=pltpu.CompilerParams(needs_layout_passes=False),
)
def _flatten_sc(x_hbm, xf_hbm, xv, fv):
    wid = lax.axis_index("s") * _NC + lax.axis_index("c")
    r0 = wid * _ROWS_W
    pltpu.sync_copy(x_hbm.at[pl.ds(r0, _ROWS_W), :], xv)

    def body(r, carry):
        i16 = lax.iota(jnp.int32, 16)
        p1 = r * _L + i16
        v1 = xv[r, pl.ds(0, 16)]
        plsc.store_scatter(fv, [p1 // _XF_C, p1 % _XF_C], v1)
        p2 = p1 + 4
        v2 = xv[r, pl.ds(4, 16)]
        plsc.store_scatter(fv, [p2 // _XF_C, p2 % _XF_C], v2, mask=i16 >= 12)
        return carry

    lax.fori_loop(0, _ROWS_W, body, 0)
    pltpu.sync_copy(fv, xf_hbm.at[pl.ds(wid * _XFR_W, _XFR_W), :])


_BLK_CUM = [(k + 1) * _XF_C // _L for k in range(_CH_XFR)]


@functools.partial(
    pl.kernel,
    mesh=_mesh,
    out_type=jax.ShapeDtypeStruct((_B, _L, _EMBED), jnp.float32),
    scratch_types=[
        pltpu.VMEM((_XFR_W, _XF_C), jnp.int32),
        pltpu.VMEM((_CH, 128), jnp.float32),
        pltpu.VMEM((_CH, _EMBED), jnp.float32),
        [pltpu.SemaphoreType.DMA] * _CH_XFR,
        pltpu.SemaphoreType.DMA,
    ],
)
def _embed_sc(xf_hbm, table_hbm, out_hbm, idx_v, rows_v, rows64_v, gsems, ssem):
    wid = lax.axis_index("s") * _NC + lax.axis_index("c")
    row0 = wid * _ROWS_W
    xfr0 = wid * _XFR_W
    pltpu.sync_copy(xf_hbm.at[pl.ds(xfr0, _XFR_W), :], idx_v)

    def body(i, carry):
        gds = [
            pltpu.async_copy(
                table_hbm.at[idx_v.at[i * _CH_XFR + k]],
                rows_v.at[pl.ds(k * _XF_C, _XF_C), :],
                gsems[k],
            )
            for k in range(_CH_XFR)
        ]
        r = row0 + i * _RBLK
        nblk = 0
        for k in range(_CH_XFR):
            gds[k].wait()

            def depad(j, c):
                for c4 in range(4):
                    rows64_v[j, pl.ds(c4 * 16, 16)] = rows_v[j, pl.ds(c4 * 16, 16)]
                return c

            lax.fori_loop(k * _XF_C, (k + 1) * _XF_C, depad, 0)
            for b in range(nblk, _BLK_CUM[k]):
                pltpu.async_copy(
                    rows64_v.at[pl.ds(b * _L, _L), :], out_hbm.at[r + b], ssem
                )
            nblk = _BLK_CUM[k]
        for b in range(_RBLK):
            pltpu.make_async_copy(
                rows64_v.at[pl.ds(b * _L, _L), :], out_hbm.at[r + b], ssem
            ).wait()
        return carry

    lax.fori_loop(0, _NCHUNK, body, 0)


def kernel(x, W):
    table = _pad_transpose_tc(W.T)
    xf = _flatten_sc(x.astype(jnp.int32))
    return _embed_sc(xf, table)


# speedup vs baseline: 3.0798x; 1.3026x over previous
import functools

import jax
import jax.numpy as jnp
from jax import lax
from jax.experimental import pallas as pl
from jax.experimental.pallas import tpu as pltpu
from jax.experimental.pallas import tpu_sc as plsc

_VOCAB = 1000000
_EMBED = 64
_B = 16384
_L = 20
_N = _B * _L
_NC = 2
_NS = 16
_NW = _NC * _NS
_ROWS_W = _B // _NW
_PER_W = _N // _NW
_XF_C = 64
_XF_R = _N // _XF_C
_XFR_W = _PER_W // _XF_C
_RBLK = 16
_CH = _RBLK * _L
_CH_XFR = _CH // _XF_C
_NCHUNK = _ROWS_W // _RBLK
_KT = 8192

_mesh = plsc.VectorSubcoreMesh(core_axis_name="c", subcore_axis_name="s")


@functools.partial(
    pl.pallas_call,
    out_shape=jax.ShapeDtypeStruct((_VOCAB, 128), jnp.float32),
    grid=(pl.cdiv(_VOCAB, _KT),),
    in_specs=[pl.BlockSpec((_EMBED, _KT), lambda i: (0, i))],
    out_specs=pl.BlockSpec((_KT, 128), lambda i: (i, 0)),
)
def _pad_transpose_tc(wt_ref, out_ref):
    t = wt_ref[...].T
    out_ref[...] = jnp.concatenate([t, jnp.zeros_like(t)], axis=1)


@functools.partial(
    pl.kernel,
    mesh=_mesh,
    out_type=jax.ShapeDtypeStruct((_XF_R, _XF_C), jnp.int32),
    scratch_types=[
        pltpu.VMEM((_ROWS_W, _L), jnp.int32),
        pltpu.VMEM((_XFR_W, _XF_C), jnp.int32),
    ],
    compiler_params=pltpu.CompilerParams(needs_layout_passes=False),
)
def _flatten_sc(x_hbm, xf_hbm, xv, fv):
    wid = lax.axis_index("s") * _NC + lax.axis_index("c")
    r0 = wid * _ROWS_W
    pltpu.sync_copy(x_hbm.at[pl.ds(r0, _ROWS_W), :], xv)

    def body(r, carry):
        i16 = lax.iota(jnp.int32, 16)
        p1 = r * _L + i16
        v1 = xv[r, pl.ds(0, 16)]
        plsc.store_scatter(fv, [p1 // _XF_C, p1 % _XF_C], v1)
        p2 = p1 + 4
        v2 = xv[r, pl.ds(4, 16)]
        plsc.store_scatter(fv, [p2 // _XF_C, p2 % _XF_C], v2, mask=i16 >= 12)
        return carry

    lax.fori_loop(0, _ROWS_W, body, 0)
    pltpu.sync_copy(fv, xf_hbm.at[pl.ds(wid * _XFR_W, _XFR_W), :])


_BLK_CUM = [(k + 1) * _XF_C // _L for k in range(_CH_XFR)]


@functools.partial(
    pl.kernel,
    mesh=_mesh,
    out_type=jax.ShapeDtypeStruct((_B, _L, _EMBED), jnp.float32),
    scratch_types=[
        pltpu.VMEM((_XFR_W, _XF_C), jnp.int32),
        pltpu.VMEM((_CH, 128), jnp.float32),
        pltpu.VMEM((_CH, _EMBED), jnp.float32),
        [pltpu.SemaphoreType.DMA] * _CH_XFR,
        pltpu.SemaphoreType.DMA,
    ],
)
def _embed_sc(xf_hbm, table_hbm, out_hbm, idx_v, rows_v, rows64_v, gsems, ssem):
    wid = lax.axis_index("s") * _NC + lax.axis_index("c")
    row0 = wid * _ROWS_W
    xfr0 = wid * _XFR_W
    pltpu.sync_copy(xf_hbm.at[pl.ds(xfr0, _XFR_W), :], idx_v)

    def body(i, carry):
        gds = [
            pltpu.async_copy(
                table_hbm.at[idx_v.at[i * _CH_XFR + k]],
                rows_v.at[pl.ds(k * _XF_C, _XF_C), :],
                gsems[k],
            )
            for k in range(_CH_XFR)
        ]
        r = row0 + i * _RBLK
        nblk = 0
        for k in range(_CH_XFR):
            gds[k].wait()

            def depad(j, c):
                for c4 in range(4):
                    rows64_v[j, pl.ds(c4 * 16, 16)] = rows_v[j, pl.ds(c4 * 16, 16)]
                return c

            lax.fori_loop(k * _XF_C, (k + 1) * _XF_C, depad, 0)
            for b in range(nblk, _BLK_CUM[k]):
                pltpu.async_copy(
                    rows64_v.at[pl.ds(b * _L, _L), :], out_hbm.at[r + b], ssem
                )
            nblk = _BLK_CUM[k]
        for b in range(_RBLK):
            pltpu.make_async_copy(
                rows64_v.at[pl.ds(b * _L, _L), :], out_hbm.at[r + b], ssem
            ).wait()
        return carry

    lax.fori_loop(0, _NCHUNK, body, 0)


def kernel(x, W):
    table = _pad_transpose_tc(W.T)
    xf = _flatten_sc(x.astype(jnp.int32))
    return _embed_sc(xf, table)


# speedup vs baseline: 3.1485x; 1.0223x over previous
import functools

import jax
import jax.numpy as jnp
from jax import lax
from jax.experimental import pallas as pl
from jax.experimental.pallas import tpu as pltpu
from jax.experimental.pallas import tpu_sc as plsc

_VOCAB = 1000000
_EMBED = 64
_B = 16384
_L = 20
_N = _B * _L
_NC = 2
_NS = 16
_NW = _NC * _NS
_ROWS_W = _B // _NW
_PER_W = _N // _NW
_XF_C = 64
_XF_R = _N // _XF_C
_XFR_W = _PER_W // _XF_C
_RBLK = 16
_CH = _RBLK * _L
_CH_XFR = _CH // _XF_C
_NCHUNK = _ROWS_W // _RBLK
_KT = 8192

_mesh = plsc.VectorSubcoreMesh(core_axis_name="c", subcore_axis_name="s")


@functools.partial(
    pl.pallas_call,
    out_shape=jax.ShapeDtypeStruct((_VOCAB, 128), jnp.float32),
    grid=(pl.cdiv(_VOCAB, _KT),),
    in_specs=[pl.BlockSpec((_EMBED, _KT), lambda i: (0, i))],
    out_specs=pl.BlockSpec((_KT, 128), lambda i: (i, 0)),
)
def _pad_transpose_tc(wt_ref, out_ref):
    t = wt_ref[...].T
    out_ref[...] = jnp.concatenate([t, jnp.zeros_like(t)], axis=1)


@functools.partial(
    pl.kernel,
    mesh=_mesh,
    out_type=jax.ShapeDtypeStruct((_XF_R, _XF_C), jnp.int32),
    scratch_types=[
        pltpu.VMEM((_ROWS_W, _L), jnp.int32),
        pltpu.VMEM((_XFR_W, _XF_C), jnp.int32),
    ],
    compiler_params=pltpu.CompilerParams(needs_layout_passes=False),
)
def _flatten_sc(x_hbm, xf_hbm, xv, fv):
    wid = lax.axis_index("s") * _NC + lax.axis_index("c")
    r0 = wid * _ROWS_W
    pltpu.sync_copy(x_hbm.at[pl.ds(r0, _ROWS_W), :], xv)

    def body(r, carry):
        i16 = lax.iota(jnp.int32, 16)
        p1 = r * _L + i16
        v1 = xv[r, pl.ds(0, 16)]
        plsc.store_scatter(fv, [p1 // _XF_C, p1 % _XF_C], v1)
        p2 = p1 + 4
        v2 = xv[r, pl.ds(4, 16)]
        plsc.store_scatter(fv, [p2 // _XF_C, p2 % _XF_C], v2, mask=i16 >= 12)
        return carry

    lax.fori_loop(0, _ROWS_W, body, 0)
    pltpu.sync_copy(fv, xf_hbm.at[pl.ds(wid * _XFR_W, _XFR_W), :])


_BLK_CUM = [(k + 1) * _XF_C // _L for k in range(_CH_XFR)]


@functools.partial(
    pl.kernel,
    mesh=_mesh,
    out_type=jax.ShapeDtypeStruct((_B, _L, _EMBED), jnp.float32),
    scratch_types=[
        pltpu.VMEM((_XFR_W, _XF_C), jnp.int32),
        pltpu.VMEM((_CH, 128), jnp.float32),
        pltpu.VMEM((_CH, _EMBED), jnp.float32),
        [pltpu.SemaphoreType.DMA] * _CH_XFR,
        pltpu.SemaphoreType.DMA,
    ],
)
def _embed_sc(xf_hbm, table_hbm, out_hbm, idx_v, rows_v, rows64_v, gsems, ssem):
    wid = lax.axis_index("s") * _NC + lax.axis_index("c")
    row0 = wid * _ROWS_W
    xfr0 = wid * _XFR_W
    pltpu.sync_copy(xf_hbm.at[pl.ds(xfr0, _XFR_W), :], idx_v)

    def fire(i, k):
        pltpu.async_copy(
            table_hbm.at[idx_v.at[i * _CH_XFR + k]],
            rows_v.at[pl.ds(k * _XF_C, _XF_C), :],
            gsems[k],
        )

    def wait_store(b):
        pltpu.make_async_copy(
            rows64_v.at[pl.ds(b * _L, _L), :], out_hbm.at[0], ssem
        ).wait()

    for k in range(_CH_XFR):
        fire(0, k)

    def body(i, carry):
        r = row0 + i * _RBLK
        nblk = 0
        for k in range(_CH_XFR):
            pltpu.make_async_copy(
                table_hbm.at[idx_v.at[i * _CH_XFR + k]],
                rows_v.at[pl.ds(k * _XF_C, _XF_C), :],
                gsems[k],
            ).wait()

            if k == 0:
                @pl.when(i > 0)
                def _drain_prev():
                    for b in range(_RBLK):
                        wait_store(b)

            def depad(j, c):
                for c4 in range(4):
                    rows64_v[j, pl.ds(c4 * 16, 16)] = rows_v[j, pl.ds(c4 * 16, 16)]
                return c

            lax.fori_loop(k * _XF_C, (k + 1) * _XF_C, depad, 0)

            @pl.when(i < _NCHUNK - 1)
            def _prefetch():
                fire(i + 1, k)

            for b in range(nblk, _BLK_CUM[k]):
                pltpu.async_copy(
                    rows64_v.at[pl.ds(b * _L, _L), :], out_hbm.at[r + b], ssem
                )
            nblk = _BLK_CUM[k]
        return carry

    lax.fori_loop(0, _NCHUNK, body, 0)
    for b in range(_RBLK):
        wait_store(b)


def kernel(x, W):
    table = _pad_transpose_tc(W.T)
    xf = _flatten_sc(x.astype(jnp.int32))
    return _embed_sc(xf, table)
